# baseline (speedup 1.0000x reference)
import functools

import jax
import jax.numpy as jnp
from jax import lax
from jax.experimental import pallas as pl
from jax.experimental.pallas import tpu as pltpu
from jax.experimental.pallas import tpu_sc as plsc

_NUM_E = 512
_DIM = 64
_G = 8
_NCHUNK = 4


def _idx_body(x_ref, e_ref, o_ref):
    ee = e_ref[...]
    es = jnp.sum(ee * ee, axis=1)[None, :] * 0.25
    for g in range(_G):
        xb = x_ref[g]

        prod2 = jax.lax.dot_general(
            xb, ee, (((0,), (1,)), ((), ())),
            preferred_element_type=jnp.float32)
        zs = jnp.sum(xb * xb, axis=0)[:, None]
        dist = (zs + es) + prod2

        m = jnp.min(dist, axis=1, keepdims=True)
        iota = jax.lax.broadcasted_iota(jnp.int32, dist.shape, 1)
        idx = jnp.min(jnp.where(dist == m, iota, _NUM_E - 1), axis=1)
        o_ref[g] = idx


def _tc_indices(xc, e2, cb):
    B, C, P = xc.shape
    CB = B // _NCHUNK
    return pl.pallas_call(
        _idx_body,
        grid=(CB // _G,),
        in_specs=[
            pl.BlockSpec((_G, C, P), lambda i, cb=cb: (i + cb * (CB // _G), 0, 0)),
            pl.BlockSpec((_NUM_E, _DIM), lambda i: (0, 0)),
        ],
        out_specs=pl.BlockSpec((_G, P), lambda i: (i, 0)),
        out_shape=jax.ShapeDtypeStruct((CB, P), jnp.int32),
        compiler_params=pltpu.CompilerParams(dimension_semantics=("arbitrary",)),
    )(xc, e2)


def _sc_gather(idx_chunk, et_flat, P):
    CB = idx_chunk.shape[0]
    info = plsc.get_sparse_core_info()
    nw = info.num_cores * info.num_subcores
    bpw = CB // nw
    mesh = plsc.VectorSubcoreMesh(core_axis_name="c", subcore_axis_name="s")

    @functools.partial(
        pl.kernel, mesh=mesh,
        out_type=jax.ShapeDtypeStruct((CB, _DIM * P), jnp.float32),
        compiler_params=pltpu.CompilerParams(needs_layout_passes=False),
        scratch_types=[
            pltpu.VMEM((_DIM * _NUM_E,), jnp.float32),
            pltpu.VMEM((P,), jnp.int32),
            pltpu.VMEM((_DIM * P,), jnp.float32),
        ],
    )
    def k(et_hbm, idx_hbm, out_hbm, et_v, idx_v, out_v):
        wid = lax.axis_index("s") * info.num_cores + lax.axis_index("c")
        pltpu.sync_copy(et_hbm, et_v)
        for bl in range(bpw):
            b = wid * bpw + bl
            pltpu.sync_copy(idx_hbm.at[b], idx_v)

            def chan(c, carry):
                def chunk(kk, carry2):
                    a = idx_v[pl.ds(kk * 16, 16)] + c * _NUM_E
                    vec = plsc.load_gather(et_v, [a])
                    out_v[pl.ds(c * P + kk * 16, 16)] = vec
                    return carry2
                return lax.fori_loop(0, P // 16, chunk, carry)

            lax.fori_loop(0, _DIM, chan, 0)
            pltpu.sync_copy(out_v, out_hbm.at[b])

    return k(et_flat, idx_chunk)


def kernel(x, e):
    B, C, H, W = x.shape
    P = H * W
    xr = x.reshape(B, C, P)
    e2 = -2.0 * e
    et_flat = e.T.reshape(-1)
    chunks = []
    for cb in range(_NCHUNK):
        idx_c = _tc_indices(xr, e2, cb)
        chunks.append(_sc_gather(idx_c, et_flat, P))
    out = jnp.concatenate(chunks, axis=0)
    return out.reshape(B, C, H, W)


# speedup vs baseline: 1.1980x; 1.1980x over previous
import functools

import jax
import jax.numpy as jnp
from jax import lax
from jax.experimental import pallas as pl
from jax.experimental.pallas import tpu as pltpu
from jax.experimental.pallas import tpu_sc as plsc

_NUM_E = 512
_DIM = 64
_G = 8
_NCHUNK = 4


def _idx_body(x_ref, e_ref, o_ref):
    ee = e_ref[...]
    es = jnp.sum(ee * ee, axis=1)[None, :] * 0.25
    for g in range(_G):
        xb = x_ref[g]

        prod2 = jax.lax.dot_general(
            xb, ee, (((0,), (1,)), ((), ())),
            preferred_element_type=jnp.float32)
        zs = jnp.sum(xb * xb, axis=0)[:, None]
        dist = (zs + es) + prod2

        m = jnp.min(dist, axis=1, keepdims=True)
        iota = jax.lax.broadcasted_iota(jnp.int32, dist.shape, 1)
        idx = jnp.min(jnp.where(dist == m, iota, _NUM_E - 1), axis=1)
        o_ref[g] = idx


def _tc_indices(xc, e2, cb):
    B, C, P = xc.shape
    CB = B // _NCHUNK
    return pl.pallas_call(
        _idx_body,
        grid=(CB // _G,),
        in_specs=[
            pl.BlockSpec((_G, C, P), lambda i, cb=cb: (i + cb * (CB // _G), 0, 0)),
            pl.BlockSpec((_NUM_E, _DIM), lambda i: (0, 0)),
        ],
        out_specs=pl.BlockSpec((_G, P), lambda i: (i, 0)),
        out_shape=jax.ShapeDtypeStruct((CB, P), jnp.int32),
        compiler_params=pltpu.CompilerParams(dimension_semantics=("arbitrary",)),
    )(xc, e2)


def _sc_gather(idx_chunk, et_flat, P):
    CB = idx_chunk.shape[0]
    info = plsc.get_sparse_core_info()
    nw = info.num_cores * info.num_subcores
    bpw = CB // nw
    mesh = plsc.VectorSubcoreMesh(core_axis_name="c", subcore_axis_name="s")

    @functools.partial(
        pl.kernel, mesh=mesh,
        out_type=jax.ShapeDtypeStruct((CB, _DIM * P), jnp.float32),
        compiler_params=pltpu.CompilerParams(needs_layout_passes=False),
        scratch_types=[
            pltpu.VMEM((_DIM * _NUM_E,), jnp.float32),
            pltpu.VMEM((P,), jnp.int32),
            pltpu.VMEM((_DIM * P,), jnp.float32),
        ],
    )
    def k(et_hbm, idx_hbm, out_hbm, et_v, idx_v, out_v):
        wid = lax.axis_index("s") * info.num_cores + lax.axis_index("c")
        pltpu.sync_copy(et_hbm, et_v)
        for bl in range(bpw):
            b = wid * bpw + bl
            pltpu.sync_copy(idx_hbm.at[b], idx_v)

            nk = P // 16

            @plsc.parallel_loop(0, _DIM * nk, unroll=8)
            def _gather_loop(i):
                c = lax.shift_right_logical(i, 6)
                kk = lax.bitwise_and(i, nk - 1)
                a = idx_v[pl.ds(kk * 16, 16)] + c * _NUM_E
                vec = plsc.load_gather(et_v, [a])
                out_v[pl.ds(i * 16, 16)] = vec
            pltpu.sync_copy(out_v, out_hbm.at[b])

    return k(et_flat, idx_chunk)


def kernel(x, e):
    B, C, H, W = x.shape
    P = H * W
    xr = x.reshape(B, C, P)
    e2 = -2.0 * e
    et_flat = e.T.reshape(-1)
    chunks = []
    for cb in range(_NCHUNK):
        idx_c = _tc_indices(xr, e2, cb)
        chunks.append(_sc_gather(idx_c, et_flat, P))
    out = jnp.concatenate(chunks, axis=0)
    return out.reshape(B, C, H, W)
